# baseline (speedup 1.0000x reference)
import functools

import jax
import jax.numpy as jnp
from jax import lax
from jax.experimental import pallas as pl
from jax.experimental.pallas import tpu as pltpu
from jax.experimental.pallas import tpu_sc as plsc

VOCAB = 1000000
HIDDEN = 64
BATCH = 4096
SEQ = 50

NC = 2
NS = 16
NW = NC * NS

TOTAL = BATCH * SEQ
ROWS_PER_W = TOTAL // NW
CHUNK = 128
NCH = ROWS_PER_W // CHUNK


def _make_kernel():
    mesh = plsc.VectorSubcoreMesh(core_axis_name="c", subcore_axis_name="s")

    @functools.partial(
        pl.kernel,
        out_type=jax.ShapeDtypeStruct((NW, NCH, CHUNK, HIDDEN), jnp.float32),
        mesh=mesh,
        scratch_types=[
            pltpu.VMEM((NCH, CHUNK), jnp.int32),
            pltpu.VMEM((CHUNK, HIDDEN), jnp.float32),
            pltpu.SemaphoreType.DMA,
        ],
        compiler_params=pltpu.CompilerParams(use_tc_tiling_on_sc=False),
    )
    def embed(ids_hbm, table_hbm, out_hbm, idx_v, rows_v, sem):
        wid = lax.axis_index("s") * NC + lax.axis_index("c")
        pltpu.sync_copy(ids_hbm.at[wid], idx_v)

        def step(j, carry):
            pltpu.async_copy(table_hbm.at[idx_v.at[j]], rows_v, sem).wait()
            pltpu.sync_copy(rows_v, out_hbm.at[wid, j])
            return carry

        lax.fori_loop(0, NCH, step, 0)

    return embed


_EMBED = _make_kernel()


def kernel(input_ids, word_embeddings):
    ids = input_ids.astype(jnp.int32).reshape(NW, NCH, CHUNK)
    out = _EMBED(ids, word_embeddings)
    return out.reshape(BATCH, SEQ, HIDDEN)


# speedup vs baseline: 1.0430x; 1.0430x over previous
import functools

import jax
import jax.numpy as jnp
from jax import lax
from jax.experimental import pallas as pl
from jax.experimental.pallas import tpu as pltpu
from jax.experimental.pallas import tpu_sc as plsc

VOCAB = 1000000
HIDDEN = 64
BATCH = 4096
SEQ = 50

NC = 2
NS = 16
NW = NC * NS

TOTAL = BATCH * SEQ
ROWS_PER_W = TOTAL // NW
CHUNK = 128
NCH = ROWS_PER_W // CHUNK
K = 5
NSC = NCH // K
SCROWS = K * CHUNK


def _make_kernel():
    mesh = plsc.VectorSubcoreMesh(core_axis_name="c", subcore_axis_name="s")

    @functools.partial(
        pl.kernel,
        out_type=jax.ShapeDtypeStruct((NW, NSC, SCROWS, HIDDEN), jnp.float32),
        mesh=mesh,
        scratch_types=[
            pltpu.VMEM((NCH, CHUNK), jnp.int32),
            pltpu.VMEM((SCROWS, HIDDEN), jnp.float32),
            pltpu.VMEM((SCROWS, HIDDEN), jnp.float32),
            pltpu.SemaphoreType.DMA,
            pltpu.SemaphoreType.DMA,
            pltpu.SemaphoreType.DMA,
            pltpu.SemaphoreType.DMA,
        ],
        compiler_params=pltpu.CompilerParams(use_tc_tiling_on_sc=False),
    )
    def embed(ids_hbm, table_hbm, out_hbm, idx_v, rows0, rows1, g0, g1, o0, o1):
        wid = lax.axis_index("s") * NC + lax.axis_index("c")
        pltpu.sync_copy(ids_hbm.at[wid], idx_v)

        bufs = (rows0, rows1)
        gsems = (g0, g1)
        osems = (o0, o1)

        def issue_gathers(s, b):
            return [
                pltpu.async_copy(
                    table_hbm.at[idx_v.at[s * K + k]],
                    bufs[b].at[pl.ds(k * CHUNK, CHUNK)],
                    gsems[b])
                for k in range(K)
            ]

        pend_g = [issue_gathers(0, 0), issue_gathers(1, 1)]
        pend_o = [None, None]
        for s in range(NSC):
            b = s % 2
            for h in pend_g[b]:
                h.wait()
            pend_o[b] = pltpu.async_copy(bufs[b], out_hbm.at[wid, s], osems[b])
            if s + 2 < NSC:
                pend_o[b].wait()
                pend_g[b] = issue_gathers(s + 2, b)
        pend_o[NSC % 2].wait()
        pend_o[(NSC - 1) % 2].wait()

    return embed


_EMBED = _make_kernel()


def kernel(input_ids, word_embeddings):
    ids = input_ids.astype(jnp.int32).reshape(NW, NCH, CHUNK)
    out = _EMBED(ids, word_embeddings)
    return out.reshape(BATCH, SEQ, HIDDEN)
